# baseline (speedup 1.0000x reference)
import functools

import jax
import jax.numpy as jnp
from jax import lax
from jax.experimental import pallas as pl
from jax.experimental.pallas import tpu as pltpu
from jax.experimental.pallas import tpu_sc as plsc

N_NODES = 10000
N_EDGES = 320000
D = 128
N_GRAPHS = 64

NC = 2
NS = 16
NW = NC * NS
EW = N_EDGES // NW
CH = 64
NFULL = EW // CH
TAIL = EW - NFULL * CH
RPT = 632
RPAD = RPT * NS

BN = 400
NB = N_NODES // BN


def _edge_aggr_body(x_hbm, src_hbm, dst_hbm, ea_hbm, out_hbm,
                    xr0, xr1, xr2, eb0, eb1, eb2, sb0, sb1, sb2,
                    db0, db1, db2, src_t, dst_t, aggr_sp,
                    sld0, sld1, sld2, ssr0, ssr1, ssr2,
                    sds0, sds1, sds2, ssc0, ssc1, ssc2):
    cid = lax.axis_index("c")
    sid = lax.axis_index("s")
    wid = cid * NS + sid
    ebase = wid * EW

    xr = (xr0, xr1, xr2)
    eb = (eb0, eb1, eb2)
    sb = (sb0, sb1, sb2)
    db = (db0, db1, db2)
    sld = (sld0, sld1, sld2)
    ssr = (ssr0, ssr1, ssr2)
    sds = (sds0, sds1, sds2)
    ssc = (ssc0, ssc1, ssc2)

    def src_sl(k):
        return src_hbm.at[pl.ds(ebase + k * CH, CH)]

    def dst_sl(k):
        return dst_hbm.at[pl.ds(ebase + k * CH, CH)]

    def ea_sl(k):
        return ea_hbm.at[pl.ds(ebase + k * CH, CH)]

    def issue_src(k, i):
        pltpu.async_copy(src_sl(k), sb[i], ssr[i])

    def wait_src(k, i):
        pltpu.make_async_copy(src_sl(k), sb[i], ssr[i]).wait()

    def issue_dst(k, i):
        pltpu.async_copy(dst_sl(k), db[i], sds[i])

    def wait_dst(k, i):
        pltpu.make_async_copy(dst_sl(k), db[i], sds[i]).wait()

    def issue_data(k, i):
        pltpu.async_copy(x_hbm.at[sb[i]], xr[i], sld[i])
        pltpu.async_copy(ea_sl(k), eb[i], sld[i])

    def wait_data(k, i):
        pltpu.make_async_copy(x_hbm.at[sb[i]], xr[i], sld[i]).wait()
        pltpu.make_async_copy(ea_sl(k), eb[i], sld[i]).wait()

    def issue_sc(i):
        pltpu.async_copy(eb[i], aggr_sp.at[db[i]], ssc[i], add=True)

    def wait_sc(i):
        pltpu.make_async_copy(eb[i], aggr_sp.at[db[i]], ssc[i]).wait()

    def compute_rows(i, n):
        def crow(r, carry):
            for j in range(4):
                w = xr[i][r, pl.ds(j * 16, 16)]
                u0 = lax.bitcast_convert_type(w << 16, jnp.float32)
                u1 = lax.bitcast_convert_type(w & jnp.int32(-65536),
                                              jnp.float32)
                s0 = pl.ds(j * 32, 16)
                s1 = pl.ds(j * 32 + 16, 16)
                eb[i][r, s0] = jnp.maximum(u0 + eb[i][r, s0], 0.0)
                eb[i][r, s1] = jnp.maximum(u1 + eb[i][r, s1], 0.0)
            return carry
        lax.fori_loop(0, n, crow, 0)

    pltpu.sync_copy(src_sl(0), sb0)
    pltpu.sync_copy(dst_sl(0), db0)
    issue_data(0, 0)
    issue_src(1, 1)
    issue_dst(1, 1)
    issue_src(2, 2)

    def zrow(r, carry):
        for j in range(8):
            eb2[r, pl.ds(j * 16, 16)] = jnp.zeros((16,), jnp.float32)
        return carry
    lax.fori_loop(0, CH, zrow, 0)
    nz = RPT // CH
    rem = RPT - nz * CH
    for k in range(nz):
        pltpu.sync_copy(eb2, aggr_sp.at[pl.ds(sid * RPT + k * CH, CH)])
    pltpu.sync_copy(eb2.at[pl.ds(0, rem)],
                    aggr_sp.at[pl.ds(sid * RPT + nz * CH, rem)])
    plsc.subcore_barrier()

    wait_src(1, 1)
    issue_data(1, 1)
    wait_data(0, 0)
    compute_rows(0, CH)
    issue_sc(0)

    wait_src(2, 2)
    issue_data(2, 2)
    issue_src(3, 0)
    issue_dst(2, 2)
    wait_data(1, 1)
    compute_rows(1, CH)
    wait_dst(1, 1)
    wait_sc(0)
    issue_sc(1)

    def steady(k, a, b, c2):
        wait_src(k + 1, b)
        issue_data(k + 1, b)
        issue_src(k + 2, c2)
        issue_dst(k + 1, b)
        wait_data(k, a)
        compute_rows(a, CH)
        wait_dst(k, a)
        wait_sc(c2)
        issue_sc(a)

    steady(2, 2, 0, 1)

    def triple(c, carry):
        k = 3 * c
        steady(k, 0, 1, 2)
        steady(k + 1, 1, 2, 0)
        steady(k + 2, 2, 0, 1)
        return carry
    lax.fori_loop(1, NFULL // 3 - 1, triple, 0)

    kl = NFULL - 3
    steady(kl, 0, 1, 2)

    et = ebase + NFULL * CH
    wait_src(kl + 2, 2)
    issue_data(kl + 2, 2)
    issue_dst(kl + 2, 2)
    pltpu.async_copy(src_hbm.at[pl.ds(et, TAIL)], src_t, ssr0)
    pltpu.async_copy(dst_hbm.at[pl.ds(et, TAIL)], dst_t, sds0)
    wait_data(kl + 1, 1)
    compute_rows(1, CH)
    wait_dst(kl + 1, 1)
    wait_sc(0)
    issue_sc(1)

    pltpu.make_async_copy(src_hbm.at[pl.ds(et, TAIL)], src_t, ssr0).wait()
    pltpu.make_async_copy(dst_hbm.at[pl.ds(et, TAIL)], dst_t, sds0).wait()
    pltpu.async_copy(x_hbm.at[src_t], xr0.at[pl.ds(0, TAIL)], sld0)
    pltpu.async_copy(ea_hbm.at[pl.ds(et, TAIL)], eb0.at[pl.ds(0, TAIL)],
                     sld0)
    wait_data(kl + 2, 2)
    compute_rows(2, CH)
    wait_dst(kl + 2, 2)
    wait_sc(1)
    issue_sc(2)

    pltpu.make_async_copy(x_hbm.at[src_t], xr0.at[pl.ds(0, TAIL)],
                          sld0).wait()
    pltpu.make_async_copy(ea_hbm.at[pl.ds(et, TAIL)],
                          eb0.at[pl.ds(0, TAIL)], sld0).wait()
    compute_rows(0, TAIL)
    wait_sc(2)
    pltpu.sync_copy(eb0.at[pl.ds(0, TAIL)], aggr_sp.at[dst_t], add=True)

    plsc.subcore_barrier()
    pltpu.sync_copy(aggr_sp.at[pl.ds(sid * RPT, RPT)],
                    out_hbm.at[cid, pl.ds(sid * RPT, RPT)])


@functools.lru_cache(maxsize=None)
def _edge_aggr_call():
    return functools.partial(
        pl.kernel,
        out_type=jax.ShapeDtypeStruct((NC, RPAD, D), jnp.float32),
        mesh=plsc.VectorSubcoreMesh(
            core_axis_name="c", subcore_axis_name="s", num_cores=NC),
        compiler_params=pltpu.CompilerParams(use_tc_tiling_on_sc=False),
        scratch_types=(
            [pltpu.VMEM((CH, D // 2), jnp.int32)] * 3
            + [pltpu.VMEM((CH, D), jnp.float32)] * 3
            + [pltpu.VMEM((CH,), jnp.int32)] * 3
            + [pltpu.VMEM((CH,), jnp.int32)] * 3
            + [pltpu.VMEM((TAIL,), jnp.int32)] * 2
            + [pltpu.VMEM_SHARED((RPAD, D), jnp.float32)]
            + [pltpu.SemaphoreType.DMA] * 12
        ),
    )(_edge_aggr_body)


def _edge_aggr(x, src, dst, ea):
    return _edge_aggr_call()(x, src, dst, ea)


def _mlp_kernel(x_ref, p_ref, w1_ref, b1_ref, w2_ref, b2_ref, o_ref):
    t = x_ref[...] + p_ref[0] + p_ref[1]
    h = jnp.maximum(
        jnp.dot(t, w1_ref[...], preferred_element_type=jnp.float32)
        + b1_ref[...], 0.0)
    h = jnp.dot(h, w2_ref[...], preferred_element_type=jnp.float32) + b2_ref[...]
    o_ref[...] = jnp.maximum(h, 0.0)


def _mlp(x, p, w1, b1, w2, b2):
    return pl.pallas_call(
        _mlp_kernel,
        grid=(NB,),
        in_specs=[
            pl.BlockSpec((BN, D), lambda i: (i, 0)),
            pl.BlockSpec((NC, BN, D), lambda i: (0, i, 0)),
            pl.BlockSpec((D, D), lambda i: (0, 0)),
            pl.BlockSpec((1, D), lambda i: (0, 0)),
            pl.BlockSpec((D, D), lambda i: (0, 0)),
            pl.BlockSpec((1, D), lambda i: (0, 0)),
        ],
        out_specs=pl.BlockSpec((BN, D), lambda i: (i, 0)),
        out_shape=jax.ShapeDtypeStruct((N_NODES, D), jnp.float32),
    )(x, p, w1, b1, w2, b2)


def _mlp_pool_kernel(x_ref, p_ref, w1_ref, b1_ref, w2_ref, b2_ref,
                     bat_ref, o_ref, sums, counts):
    i = pl.program_id(0)
    t = x_ref[...] + p_ref[0] + p_ref[1]
    h = jnp.maximum(
        jnp.dot(t, w1_ref[...], preferred_element_type=jnp.float32)
        + b1_ref[...], 0.0)
    h = jnp.dot(h, w2_ref[...], preferred_element_type=jnp.float32) + b2_ref[...]
    h = jnp.maximum(h, 0.0)

    bb = bat_ref[...].reshape(1, BN)
    onehot = (lax.broadcasted_iota(jnp.int32, (N_GRAPHS, BN), 0)
              == jnp.broadcast_to(bb, (N_GRAPHS, BN))).astype(jnp.float32)
    part = jnp.dot(onehot, h, preferred_element_type=jnp.float32)
    cnt = jnp.broadcast_to(jnp.sum(onehot, axis=1, keepdims=True),
                           (N_GRAPHS, D))

    @pl.when(i == 0)
    def _():
        sums[...] = part
        counts[...] = cnt

    @pl.when(i > 0)
    def _():
        sums[...] = sums[...] + part
        counts[...] = counts[...] + cnt

    @pl.when(i == NB - 1)
    def _():
        o_ref[...] = sums[...] / jnp.maximum(counts[...], 1.0)


def _mlp_pool(x, p, w1, b1, w2, b2, bat3):
    return pl.pallas_call(
        _mlp_pool_kernel,
        grid=(NB,),
        in_specs=[
            pl.BlockSpec((BN, D), lambda i: (i, 0)),
            pl.BlockSpec((NC, BN, D), lambda i: (0, i, 0)),
            pl.BlockSpec((D, D), lambda i: (0, 0)),
            pl.BlockSpec((1, D), lambda i: (0, 0)),
            pl.BlockSpec((D, D), lambda i: (0, 0)),
            pl.BlockSpec((1, D), lambda i: (0, 0)),
            pl.BlockSpec((1, 1, BN), lambda i: (i, 0, 0)),
        ],
        out_specs=pl.BlockSpec((N_GRAPHS, D), lambda i: (0, 0)),
        out_shape=jax.ShapeDtypeStruct((N_GRAPHS, D), jnp.float32),
        scratch_shapes=[
            pltpu.VMEM((N_GRAPHS, D), jnp.float32),
            pltpu.VMEM((N_GRAPHS, D), jnp.float32),
        ],
    )(x, p, w1, b1, w2, b2, bat3)


def _perm_bf16(v):
    vb = v.astype(jnp.bfloat16).reshape(
        N_NODES, D // 32, 2, 16).transpose(0, 1, 3, 2)
    return jax.lax.bitcast_convert_type(
        vb.reshape(N_NODES, D // 2, 2), jnp.int32)


@jax.jit
def kernel(x, edge_index, edge_attr, batch, W1, b1, W2, b2):
    src = edge_index[0].astype(jnp.int32)
    dst = edge_index[1].astype(jnp.int32)
    b1r = b1.reshape(1, D)
    b2r = b2.reshape(1, D)
    bat3 = batch.astype(jnp.int32).reshape(NB, 1, BN)

    p = _edge_aggr(_perm_bf16(x), src, dst, edge_attr)
    h1 = _mlp(x, p, W1, b1r, W2, b2r)
    p2 = _edge_aggr(_perm_bf16(h1), src, dst, edge_attr)
    return _mlp_pool(h1, p2, W1, b1r, W2, b2r, bat3)


# speedup vs baseline: 1.4331x; 1.4331x over previous
import functools

import jax
import jax.numpy as jnp
from jax import lax
from jax.experimental import pallas as pl
from jax.experimental.pallas import tpu as pltpu
from jax.experimental.pallas import tpu_sc as plsc

N_NODES = 10000
N_EDGES = 320000
D = 128
N_GRAPHS = 64

NC = 2
NS = 16
NW = NC * NS
EW = N_EDGES // NW
CH = 64
NFULL = EW // CH
TAIL = EW - NFULL * CH
RPT = 632
RPAD = RPT * NS

BN = 400
NB = N_NODES // BN


def _edge_aggr_body(x_hbm, src_hbm, dst_hbm, ea_hbm, out_hbm,
                    xr0, xr1, xr2, eb0, eb1, eb2, sb0, sb1, sb2,
                    db0, db1, db2, src_t, dst_t, aggr_sp,
                    sld0, sld1, sld2, ssr0, ssr1, ssr2,
                    sds0, sds1, sds2, ssc0, ssc1, ssc2):
    cid = lax.axis_index("c")
    sid = lax.axis_index("s")
    wid = cid * NS + sid
    ebase = wid * EW

    xr = (xr0, xr1, xr2)
    eb = (eb0, eb1, eb2)
    sb = (sb0, sb1, sb2)
    db = (db0, db1, db2)
    sld = (sld0, sld1, sld2)
    ssr = (ssr0, ssr1, ssr2)
    sds = (sds0, sds1, sds2)
    ssc = (ssc0, ssc1, ssc2)

    def src_sl(k):
        return src_hbm.at[pl.ds(ebase + k * CH, CH)]

    def dst_sl(k):
        return dst_hbm.at[pl.ds(ebase + k * CH, CH)]

    def ea_sl(k):
        return ea_hbm.at[pl.ds(ebase + k * CH, CH)]

    def issue_src(k, i):
        pltpu.async_copy(src_sl(k), sb[i], ssr[i])

    def wait_src(k, i):
        pltpu.make_async_copy(src_sl(k), sb[i], ssr[i]).wait()

    def issue_dst(k, i):
        pltpu.async_copy(dst_sl(k), db[i], sds[i])

    def wait_dst(k, i):
        pltpu.make_async_copy(dst_sl(k), db[i], sds[i]).wait()

    def issue_data(k, i):
        pltpu.async_copy(x_hbm.at[sb[i]], xr[i], sld[i])
        pltpu.async_copy(ea_sl(k), eb[i], sld[i])

    def wait_data(k, i):
        pltpu.make_async_copy(x_hbm.at[sb[i]], xr[i], sld[i]).wait()
        pltpu.make_async_copy(ea_sl(k), eb[i], sld[i]).wait()

    def issue_sc(i):
        pltpu.async_copy(xr[i], aggr_sp.at[db[i]], ssc[i], add=True)

    def wait_sc(i):
        pltpu.make_async_copy(xr[i], aggr_sp.at[db[i]], ssc[i]).wait()

    def compute_rows(i, n):
        def crow(r, carry):
            for j in range(8):
                sl = pl.ds(j * 16, 16)
                xr[i][r, sl] = jnp.maximum(
                    xr[i][r, sl] + eb[i][r, sl], 0.0)
            return carry
        lax.fori_loop(0, n, crow, 0)

    pltpu.sync_copy(src_sl(0), sb0)
    pltpu.sync_copy(dst_sl(0), db0)
    issue_data(0, 0)
    issue_src(1, 1)
    issue_dst(1, 1)
    issue_src(2, 2)

    def zrow(r, carry):
        for j in range(8):
            eb2[r, pl.ds(j * 16, 16)] = jnp.zeros((16,), jnp.float32)
        return carry
    lax.fori_loop(0, CH, zrow, 0)
    nz = RPT // CH
    rem = RPT - nz * CH
    for k in range(nz):
        pltpu.sync_copy(eb2, aggr_sp.at[pl.ds(sid * RPT + k * CH, CH)])
    pltpu.sync_copy(eb2.at[pl.ds(0, rem)],
                    aggr_sp.at[pl.ds(sid * RPT + nz * CH, rem)])
    plsc.subcore_barrier()

    wait_src(1, 1)
    issue_data(1, 1)
    wait_data(0, 0)
    compute_rows(0, CH)
    issue_sc(0)

    wait_src(2, 2)
    issue_data(2, 2)
    issue_src(3, 0)
    issue_dst(2, 2)
    wait_data(1, 1)
    compute_rows(1, CH)
    wait_dst(1, 1)
    wait_sc(0)
    issue_sc(1)

    def steady(k, a, b, c2):
        wait_src(k + 1, b)
        issue_data(k + 1, b)
        issue_src(k + 2, c2)
        issue_dst(k + 1, b)
        wait_data(k, a)
        compute_rows(a, CH)
        wait_dst(k, a)
        wait_sc(c2)
        issue_sc(a)

    steady(2, 2, 0, 1)

    def triple(c, carry):
        k = 3 * c
        steady(k, 0, 1, 2)
        steady(k + 1, 1, 2, 0)
        steady(k + 2, 2, 0, 1)
        return carry
    lax.fori_loop(1, NFULL // 3 - 1, triple, 0)

    kl = NFULL - 3
    steady(kl, 0, 1, 2)

    et = ebase + NFULL * CH
    wait_src(kl + 2, 2)
    issue_data(kl + 2, 2)
    issue_dst(kl + 2, 2)
    pltpu.async_copy(src_hbm.at[pl.ds(et, TAIL)], src_t, ssr0)
    pltpu.async_copy(dst_hbm.at[pl.ds(et, TAIL)], dst_t, sds0)
    wait_data(kl + 1, 1)
    compute_rows(1, CH)
    wait_dst(kl + 1, 1)
    wait_sc(0)
    issue_sc(1)

    pltpu.make_async_copy(src_hbm.at[pl.ds(et, TAIL)], src_t, ssr0).wait()
    pltpu.make_async_copy(dst_hbm.at[pl.ds(et, TAIL)], dst_t, sds0).wait()
    pltpu.async_copy(x_hbm.at[src_t], xr0.at[pl.ds(0, TAIL)], sld0)
    pltpu.async_copy(ea_hbm.at[pl.ds(et, TAIL)], eb0.at[pl.ds(0, TAIL)],
                     sld0)
    wait_data(kl + 2, 2)
    compute_rows(2, CH)
    wait_dst(kl + 2, 2)
    wait_sc(1)
    issue_sc(2)

    pltpu.make_async_copy(x_hbm.at[src_t], xr0.at[pl.ds(0, TAIL)],
                          sld0).wait()
    pltpu.make_async_copy(ea_hbm.at[pl.ds(et, TAIL)],
                          eb0.at[pl.ds(0, TAIL)], sld0).wait()
    compute_rows(0, TAIL)
    wait_sc(2)
    pltpu.sync_copy(xr0.at[pl.ds(0, TAIL)], aggr_sp.at[dst_t], add=True)

    plsc.subcore_barrier()
    pltpu.sync_copy(aggr_sp.at[pl.ds(sid * RPT, RPT)],
                    out_hbm.at[cid, pl.ds(sid * RPT, RPT)])


@functools.lru_cache(maxsize=None)
def _edge_aggr_call():
    return functools.partial(
        pl.kernel,
        out_type=jax.ShapeDtypeStruct((NC, RPAD, D), jnp.float32),
        mesh=plsc.VectorSubcoreMesh(
            core_axis_name="c", subcore_axis_name="s", num_cores=NC),
        scratch_types=(
            [pltpu.VMEM((CH, D), jnp.float32)] * 3
            + [pltpu.VMEM((CH, D), jnp.float32)] * 3
            + [pltpu.VMEM((CH,), jnp.int32)] * 3
            + [pltpu.VMEM((CH,), jnp.int32)] * 3
            + [pltpu.VMEM((TAIL,), jnp.int32)] * 2
            + [pltpu.VMEM_SHARED((RPAD, D), jnp.float32)]
            + [pltpu.SemaphoreType.DMA] * 12
        ),
    )(_edge_aggr_body)


def _edge_aggr(x, src, dst, ea):
    return _edge_aggr_call()(x, src, dst, ea)


def _mlp_kernel(x_ref, p_ref, w1_ref, b1_ref, w2_ref, b2_ref, o_ref):
    t = x_ref[...] + p_ref[0] + p_ref[1]
    h = jnp.maximum(
        jnp.dot(t, w1_ref[...], preferred_element_type=jnp.float32)
        + b1_ref[...], 0.0)
    h = jnp.dot(h, w2_ref[...], preferred_element_type=jnp.float32) + b2_ref[...]
    o_ref[...] = jnp.maximum(h, 0.0)


def _mlp(x, p, w1, b1, w2, b2):
    return pl.pallas_call(
        _mlp_kernel,
        grid=(NB,),
        in_specs=[
            pl.BlockSpec((BN, D), lambda i: (i, 0)),
            pl.BlockSpec((NC, BN, D), lambda i: (0, i, 0)),
            pl.BlockSpec((D, D), lambda i: (0, 0)),
            pl.BlockSpec((1, D), lambda i: (0, 0)),
            pl.BlockSpec((D, D), lambda i: (0, 0)),
            pl.BlockSpec((1, D), lambda i: (0, 0)),
        ],
        out_specs=pl.BlockSpec((BN, D), lambda i: (i, 0)),
        out_shape=jax.ShapeDtypeStruct((N_NODES, D), jnp.float32),
    )(x, p, w1, b1, w2, b2)


def _mlp_pool_kernel(x_ref, p_ref, w1_ref, b1_ref, w2_ref, b2_ref,
                     bat_ref, o_ref, sums, counts):
    i = pl.program_id(0)
    t = x_ref[...] + p_ref[0] + p_ref[1]
    h = jnp.maximum(
        jnp.dot(t, w1_ref[...], preferred_element_type=jnp.float32)
        + b1_ref[...], 0.0)
    h = jnp.dot(h, w2_ref[...], preferred_element_type=jnp.float32) + b2_ref[...]
    h = jnp.maximum(h, 0.0)

    bb = bat_ref[...].reshape(1, BN)
    onehot = (lax.broadcasted_iota(jnp.int32, (N_GRAPHS, BN), 0)
              == jnp.broadcast_to(bb, (N_GRAPHS, BN))).astype(jnp.float32)
    part = jnp.dot(onehot, h, preferred_element_type=jnp.float32)
    cnt = jnp.broadcast_to(jnp.sum(onehot, axis=1, keepdims=True),
                           (N_GRAPHS, D))

    @pl.when(i == 0)
    def _():
        sums[...] = part
        counts[...] = cnt

    @pl.when(i > 0)
    def _():
        sums[...] = sums[...] + part
        counts[...] = counts[...] + cnt

    @pl.when(i == NB - 1)
    def _():
        o_ref[...] = sums[...] / jnp.maximum(counts[...], 1.0)


def _mlp_pool(x, p, w1, b1, w2, b2, bat3):
    return pl.pallas_call(
        _mlp_pool_kernel,
        grid=(NB,),
        in_specs=[
            pl.BlockSpec((BN, D), lambda i: (i, 0)),
            pl.BlockSpec((NC, BN, D), lambda i: (0, i, 0)),
            pl.BlockSpec((D, D), lambda i: (0, 0)),
            pl.BlockSpec((1, D), lambda i: (0, 0)),
            pl.BlockSpec((D, D), lambda i: (0, 0)),
            pl.BlockSpec((1, D), lambda i: (0, 0)),
            pl.BlockSpec((1, 1, BN), lambda i: (i, 0, 0)),
        ],
        out_specs=pl.BlockSpec((N_GRAPHS, D), lambda i: (0, 0)),
        out_shape=jax.ShapeDtypeStruct((N_GRAPHS, D), jnp.float32),
        scratch_shapes=[
            pltpu.VMEM((N_GRAPHS, D), jnp.float32),
            pltpu.VMEM((N_GRAPHS, D), jnp.float32),
        ],
    )(x, p, w1, b1, w2, b2, bat3)


@jax.jit
def kernel(x, edge_index, edge_attr, batch, W1, b1, W2, b2):
    src = edge_index[0].astype(jnp.int32)
    dst = edge_index[1].astype(jnp.int32)
    b1r = b1.reshape(1, D)
    b2r = b2.reshape(1, D)
    bat3 = batch.astype(jnp.int32).reshape(NB, 1, BN)

    p = _edge_aggr(x, src, dst, edge_attr)
    h1 = _mlp(x, p, W1, b1r, W2, b2r)
    p2 = _edge_aggr(h1, src, dst, edge_attr)
    return _mlp_pool(h1, p2, W1, b1r, W2, b2r, bat3)


# speedup vs baseline: 1.4531x; 1.0139x over previous
import functools

import jax
import jax.numpy as jnp
from jax import lax
from jax.experimental import pallas as pl
from jax.experimental.pallas import tpu as pltpu
from jax.experimental.pallas import tpu_sc as plsc

N_NODES = 10000
N_EDGES = 320000
D = 128
N_GRAPHS = 64

NC = 2
NS = 16
NW = NC * NS
EW = N_EDGES // NW
CH = 64
NFULL = EW // CH
TAIL = EW - NFULL * CH
RPT = 632
RPAD = RPT * NS

BN = 400
NB = N_NODES // BN


def _edge_aggr_body(x_hbm, src_hbm, dst_hbm, ea_hbm, out_hbm,
                    xr0, xr1, xr2, eb0, eb1, eb2, sb0, sb1, sb2,
                    db0, db1, db2, src_t, dst_t, aggr_sp,
                    sld0, sld1, sld2, ssr0, ssr1, ssr2,
                    sds0, sds1, sds2, ssc0, ssc1, ssc2):
    cid = lax.axis_index("c")
    sid = lax.axis_index("s")
    wid = cid * NS + sid
    ebase = wid * EW

    xr = (xr0, xr1, xr2)
    eb = (eb0, eb1, eb2)
    sb = (sb0, sb1, sb2)
    db = (db0, db1, db2)
    sld = (sld0, sld1, sld2)
    ssr = (ssr0, ssr1, ssr2)
    sds = (sds0, sds1, sds2)
    ssc = (ssc0, ssc1, ssc2)

    def src_sl(k):
        return src_hbm.at[pl.ds(ebase + k * CH, CH)]

    def dst_sl(k):
        return dst_hbm.at[pl.ds(ebase + k * CH, CH)]

    def ea_sl(k):
        return ea_hbm.at[pl.ds(ebase + k * CH, CH)]

    def issue_src(k, i):
        pltpu.async_copy(src_sl(k), sb[i], ssr[i])

    def wait_src(k, i):
        pltpu.make_async_copy(src_sl(k), sb[i], ssr[i]).wait()

    def issue_dst(k, i):
        pltpu.async_copy(dst_sl(k), db[i], sds[i])

    def wait_dst(k, i):
        pltpu.make_async_copy(dst_sl(k), db[i], sds[i]).wait()

    def issue_data(k, i):
        pltpu.async_copy(x_hbm.at[sb[i]], xr[i], sld[i])
        pltpu.async_copy(ea_sl(k), eb[i], sld[i])

    def wait_data(k, i):
        pltpu.make_async_copy(x_hbm.at[sb[i]], xr[i], sld[i]).wait()
        pltpu.make_async_copy(ea_sl(k), eb[i], sld[i]).wait()

    def issue_sc(i):
        pltpu.async_copy(xr[i], aggr_sp.at[db[i]], ssc[i], add=True)

    def wait_sc(i):
        pltpu.make_async_copy(xr[i], aggr_sp.at[db[i]], ssc[i]).wait()

    def compute_rows(i, n):
        def crow(h, carry):
            r = h * 2
            for rr in (0, 1):
                for j in range(8):
                    sl = pl.ds(j * 16, 16)
                    xr[i][r + rr, sl] = jnp.maximum(
                        xr[i][r + rr, sl] + eb[i][r + rr, sl], 0.0)
            return carry
        lax.fori_loop(0, n // 2, crow, 0)

    pltpu.sync_copy(src_sl(0), sb0)
    pltpu.sync_copy(dst_sl(0), db0)
    issue_data(0, 0)
    issue_src(1, 1)
    issue_dst(1, 1)
    issue_src(2, 2)

    def zrow(r, carry):
        for j in range(8):
            eb2[r, pl.ds(j * 16, 16)] = jnp.zeros((16,), jnp.float32)
        return carry
    lax.fori_loop(0, CH, zrow, 0)
    nz = RPT // CH
    rem = RPT - nz * CH
    for k in range(nz):
        pltpu.sync_copy(eb2, aggr_sp.at[pl.ds(sid * RPT + k * CH, CH)])
    pltpu.sync_copy(eb2.at[pl.ds(0, rem)],
                    aggr_sp.at[pl.ds(sid * RPT + nz * CH, rem)])
    plsc.subcore_barrier()

    wait_src(1, 1)
    issue_data(1, 1)
    wait_data(0, 0)
    compute_rows(0, CH)
    issue_sc(0)

    wait_src(2, 2)
    issue_data(2, 2)
    issue_src(3, 0)
    issue_dst(2, 2)
    wait_data(1, 1)
    compute_rows(1, CH)
    wait_dst(1, 1)
    wait_sc(0)
    issue_sc(1)

    def steady(k, a, b, c2):
        wait_src(k + 1, b)
        issue_data(k + 1, b)
        issue_src(k + 2, c2)
        issue_dst(k + 1, b)
        wait_data(k, a)
        compute_rows(a, CH)
        wait_dst(k, a)
        wait_sc(c2)
        issue_sc(a)

    steady(2, 2, 0, 1)

    def triple(c, carry):
        k = 3 * c
        steady(k, 0, 1, 2)
        steady(k + 1, 1, 2, 0)
        steady(k + 2, 2, 0, 1)
        return carry
    lax.fori_loop(1, NFULL // 3 - 1, triple, 0)

    kl = NFULL - 3
    steady(kl, 0, 1, 2)

    et = ebase + NFULL * CH
    wait_src(kl + 2, 2)
    issue_data(kl + 2, 2)
    issue_dst(kl + 2, 2)
    pltpu.async_copy(src_hbm.at[pl.ds(et, TAIL)], src_t, ssr0)
    pltpu.async_copy(dst_hbm.at[pl.ds(et, TAIL)], dst_t, sds0)
    wait_data(kl + 1, 1)
    compute_rows(1, CH)
    wait_dst(kl + 1, 1)
    wait_sc(0)
    issue_sc(1)

    pltpu.make_async_copy(src_hbm.at[pl.ds(et, TAIL)], src_t, ssr0).wait()
    pltpu.make_async_copy(dst_hbm.at[pl.ds(et, TAIL)], dst_t, sds0).wait()
    pltpu.async_copy(x_hbm.at[src_t], xr0.at[pl.ds(0, TAIL)], sld0)
    pltpu.async_copy(ea_hbm.at[pl.ds(et, TAIL)], eb0.at[pl.ds(0, TAIL)],
                     sld0)
    wait_data(kl + 2, 2)
    compute_rows(2, CH)
    wait_dst(kl + 2, 2)
    wait_sc(1)
    issue_sc(2)

    pltpu.make_async_copy(x_hbm.at[src_t], xr0.at[pl.ds(0, TAIL)],
                          sld0).wait()
    pltpu.make_async_copy(ea_hbm.at[pl.ds(et, TAIL)],
                          eb0.at[pl.ds(0, TAIL)], sld0).wait()
    compute_rows(0, TAIL)
    wait_sc(2)
    pltpu.sync_copy(xr0.at[pl.ds(0, TAIL)], aggr_sp.at[dst_t], add=True)

    plsc.subcore_barrier()
    pltpu.sync_copy(aggr_sp.at[pl.ds(sid * RPT, RPT)],
                    out_hbm.at[cid, pl.ds(sid * RPT, RPT)])


@functools.lru_cache(maxsize=None)
def _edge_aggr_call():
    return functools.partial(
        pl.kernel,
        out_type=jax.ShapeDtypeStruct((NC, RPAD, D), jnp.float32),
        mesh=plsc.VectorSubcoreMesh(
            core_axis_name="c", subcore_axis_name="s", num_cores=NC),
        scratch_types=(
            [pltpu.VMEM((CH, D), jnp.float32)] * 3
            + [pltpu.VMEM((CH, D), jnp.float32)] * 3
            + [pltpu.VMEM((CH,), jnp.int32)] * 3
            + [pltpu.VMEM((CH,), jnp.int32)] * 3
            + [pltpu.VMEM((TAIL,), jnp.int32)] * 2
            + [pltpu.VMEM_SHARED((RPAD, D), jnp.float32)]
            + [pltpu.SemaphoreType.DMA] * 12
        ),
    )(_edge_aggr_body)


def _edge_aggr(x, src, dst, ea):
    return _edge_aggr_call()(x, src, dst, ea)


def _mlp_kernel(x_ref, p_ref, w1_ref, b1_ref, w2_ref, b2_ref, o_ref):
    t = x_ref[...] + p_ref[0] + p_ref[1]
    h = jnp.maximum(
        jnp.dot(t, w1_ref[...], preferred_element_type=jnp.float32)
        + b1_ref[...], 0.0)
    h = jnp.dot(h, w2_ref[...], preferred_element_type=jnp.float32) + b2_ref[...]
    o_ref[...] = jnp.maximum(h, 0.0)


def _mlp(x, p, w1, b1, w2, b2):
    return pl.pallas_call(
        _mlp_kernel,
        grid=(NB,),
        in_specs=[
            pl.BlockSpec((BN, D), lambda i: (i, 0)),
            pl.BlockSpec((NC, BN, D), lambda i: (0, i, 0)),
            pl.BlockSpec((D, D), lambda i: (0, 0)),
            pl.BlockSpec((1, D), lambda i: (0, 0)),
            pl.BlockSpec((D, D), lambda i: (0, 0)),
            pl.BlockSpec((1, D), lambda i: (0, 0)),
        ],
        out_specs=pl.BlockSpec((BN, D), lambda i: (i, 0)),
        out_shape=jax.ShapeDtypeStruct((N_NODES, D), jnp.float32),
    )(x, p, w1, b1, w2, b2)


def _mlp_pool_kernel(x_ref, p_ref, w1_ref, b1_ref, w2_ref, b2_ref,
                     bat_ref, o_ref, sums, counts):
    i = pl.program_id(0)
    t = x_ref[...] + p_ref[0] + p_ref[1]
    h = jnp.maximum(
        jnp.dot(t, w1_ref[...], preferred_element_type=jnp.float32)
        + b1_ref[...], 0.0)
    h = jnp.dot(h, w2_ref[...], preferred_element_type=jnp.float32) + b2_ref[...]
    h = jnp.maximum(h, 0.0)

    bb = bat_ref[...].reshape(1, BN)
    onehot = (lax.broadcasted_iota(jnp.int32, (N_GRAPHS, BN), 0)
              == jnp.broadcast_to(bb, (N_GRAPHS, BN))).astype(jnp.float32)
    part = jnp.dot(onehot, h, preferred_element_type=jnp.float32)
    cnt = jnp.broadcast_to(jnp.sum(onehot, axis=1, keepdims=True),
                           (N_GRAPHS, D))

    @pl.when(i == 0)
    def _():
        sums[...] = part
        counts[...] = cnt

    @pl.when(i > 0)
    def _():
        sums[...] = sums[...] + part
        counts[...] = counts[...] + cnt

    @pl.when(i == NB - 1)
    def _():
        o_ref[...] = sums[...] / jnp.maximum(counts[...], 1.0)


def _mlp_pool(x, p, w1, b1, w2, b2, bat3):
    return pl.pallas_call(
        _mlp_pool_kernel,
        grid=(NB,),
        in_specs=[
            pl.BlockSpec((BN, D), lambda i: (i, 0)),
            pl.BlockSpec((NC, BN, D), lambda i: (0, i, 0)),
            pl.BlockSpec((D, D), lambda i: (0, 0)),
            pl.BlockSpec((1, D), lambda i: (0, 0)),
            pl.BlockSpec((D, D), lambda i: (0, 0)),
            pl.BlockSpec((1, D), lambda i: (0, 0)),
            pl.BlockSpec((1, 1, BN), lambda i: (i, 0, 0)),
        ],
        out_specs=pl.BlockSpec((N_GRAPHS, D), lambda i: (0, 0)),
        out_shape=jax.ShapeDtypeStruct((N_GRAPHS, D), jnp.float32),
        scratch_shapes=[
            pltpu.VMEM((N_GRAPHS, D), jnp.float32),
            pltpu.VMEM((N_GRAPHS, D), jnp.float32),
        ],
    )(x, p, w1, b1, w2, b2, bat3)


@jax.jit
def kernel(x, edge_index, edge_attr, batch, W1, b1, W2, b2):
    src = edge_index[0].astype(jnp.int32)
    dst = edge_index[1].astype(jnp.int32)
    b1r = b1.reshape(1, D)
    b2r = b2.reshape(1, D)
    bat3 = batch.astype(jnp.int32).reshape(NB, 1, BN)

    p = _edge_aggr(x, src, dst, edge_attr)
    h1 = _mlp(x, p, W1, b1r, W2, b2r)
    p2 = _edge_aggr(h1, src, dst, edge_attr)
    return _mlp_pool(h1, p2, W1, b1r, W2, b2r, bat3)
